# baseline (speedup 1.0000x reference)
import jax
import jax.numpy as jnp
from jax.experimental import pallas as pl
from jax.experimental.pallas import tpu as pltpu

S = 32768
A = 1024
N = 256
PREV0 = S - 2 * N
TERM0 = S - N
NCHUNK = 8
CHUNK = TERM0 // NCHUNK


def _body(x_ref, d_ref, o_ref, tbuf, csem, tsem_in, tsem_out):
    copies = []
    for k in range(NCHUNK):
        c = pltpu.make_async_copy(
            x_ref.at[pl.ds(k * CHUNK, CHUNK), :],
            o_ref.at[pl.ds(k * CHUNK, CHUNK), :],
            csem.at[k],
        )
        c.start()
        copies.append(c)
    tin = pltpu.make_async_copy(x_ref.at[pl.ds(PREV0, N), :], tbuf, tsem_in)
    tin.start()
    tin.wait()
    tbuf[...] = tbuf[...] + d_ref[...]
    tout = pltpu.make_async_copy(tbuf, o_ref.at[pl.ds(TERM0, N), :], tsem_out)
    tout.start()
    tout.wait()
    for c in copies:
        c.wait()


def kernel(expected, drift):
    drift2d = drift.reshape(N, 1)
    return pl.pallas_call(
        _body,
        in_specs=[
            pl.BlockSpec(memory_space=pltpu.MemorySpace.HBM),
            pl.BlockSpec(memory_space=pltpu.MemorySpace.VMEM),
        ],
        out_specs=pl.BlockSpec(memory_space=pltpu.MemorySpace.HBM),
        out_shape=jax.ShapeDtypeStruct((S, A), expected.dtype),
        scratch_shapes=[
            pltpu.VMEM((N, A), jnp.float32),
            pltpu.SemaphoreType.DMA((NCHUNK,)),
            pltpu.SemaphoreType.DMA,
            pltpu.SemaphoreType.DMA,
        ],
    )(expected, drift2d)


# speedup vs baseline: 47.9513x; 47.9513x over previous
import jax
import jax.numpy as jnp
from jax.experimental import pallas as pl
from jax.experimental.pallas import tpu as pltpu

S = 32768
A = 1024
N = 256
B = 2048
M = 4
NCH = S // B


def _body(x_ref, d_ref, o_ref, *rest):
    bufs = rest[:M]
    isem, osem = rest[M], rest[M + 1]
    cins = [None] * NCH
    couts = [None] * NCH

    def start_in(i):
        b = i % M
        c = pltpu.make_async_copy(
            x_ref.at[pl.ds(i * B, B), :], bufs[b], isem.at[b])
        c.start()
        cins[i] = c

    for i in range(M):
        start_in(i)
    for i in range(NCH):
        b = i % M
        cins[i].wait()
        if i == NCH - 1:
            bufs[b][B - N:B, :] = bufs[b][B - 2 * N:B - N, :] + d_ref[...]
        c = pltpu.make_async_copy(
            bufs[b], o_ref.at[pl.ds(i * B, B), :], osem.at[b])
        c.start()
        couts[i] = c
        if i + M < NCH:
            couts[i].wait()
            start_in(i + M)
    for i in range(NCH - M, NCH):
        couts[i].wait()


def kernel(expected, drift):
    drift2d = drift.reshape(N, 1)
    return pl.pallas_call(
        _body,
        in_specs=[
            pl.BlockSpec(memory_space=pltpu.MemorySpace.HBM),
            pl.BlockSpec(memory_space=pltpu.MemorySpace.VMEM),
        ],
        out_specs=pl.BlockSpec(memory_space=pltpu.MemorySpace.HBM),
        out_shape=jax.ShapeDtypeStruct((S, A), expected.dtype),
        scratch_shapes=(
            [pltpu.VMEM((B, A), jnp.float32) for _ in range(M)]
            + [pltpu.SemaphoreType.DMA((M,)), pltpu.SemaphoreType.DMA((M,))]
        ),
    )(expected, drift2d)
